# Initial kernel scaffold; baseline (speedup 1.0000x reference)
import jax
import jax.numpy as jnp
from jax.experimental import pallas as pl


def kernel(x, node_indices, src, tgt, W_qkv, b_qkv, W1, b1, W2, b2, ln1_g, ln1_b, ln2_g, ln2_b):
    raise NotImplementedError("write your pallas kernel here")


# speedup vs baseline: 25.8368x; 25.8368x over previous
import functools

import jax
import jax.numpy as jnp
from jax import lax
from jax.experimental import pallas as pl
from jax.experimental.pallas import tpu as pltpu
from jax.experimental.pallas import tpu_sc as plsc

N = 10000
E = 320000
D = 128
H = 8
DH = 16

NC = 2
NS = 16
NW = NC * NS
EPW = E // NW
C = 80
NCHUNK = EPW // C
NPS = N // NS
DRA = 632
DRB = N - (NS - 1) * DRA
BN = 1000


def _qkv_body(x_ref, wt_ref, b_ref, q_ref, k_ref, v_ref):
    acc = jnp.dot(x_ref[...], wt_ref[...], preferred_element_type=jnp.float32)
    acc = acc + b_ref[...]
    q_ref[...] = acc[:, 0 * D:1 * D]
    k_ref[...] = acc[:, 1 * D:2 * D]
    v_ref[...] = acc[:, 2 * D:3 * D]


def _qkv_project(x2, wt, b):
    grid = N // BN
    out = jax.ShapeDtypeStruct((N, D), jnp.float32)
    return pl.pallas_call(
        _qkv_body,
        grid=(grid,),
        in_specs=[
            pl.BlockSpec((BN, D), lambda i: (i, 0)),
            pl.BlockSpec((D, 3 * D), lambda i: (0, 0)),
            pl.BlockSpec((1, 3 * D), lambda i: (0, 0)),
        ],
        out_specs=[
            pl.BlockSpec((BN, D), lambda i: (i, 0)),
            pl.BlockSpec((BN, D), lambda i: (i, 0)),
            pl.BlockSpec((BN, D), lambda i: (i, 0)),
        ],
        out_shape=[out, out, out],
    )(x2, wt, b)


def _edge_body(q_hbm, k_hbm, v_hbm, src_hbm, tgt_hbm,
               wn_out, vs_out,
               sidx, tidx, qbuf, kbuf, vbuf, zbuf,
               wn_sh, vs_sh, sem):
    cid = lax.axis_index("c")
    sid = lax.axis_index("s")
    zero16 = jnp.zeros((16,), jnp.float32)

    def _z_row(i, _):
        for j in range(D // 16):
            qbuf[i, pl.ds(j * 16, 16)] = zero16
        zbuf[i, :] = zero16
        return 0
    lax.fori_loop(0, C, _z_row, 0)

    for t in range((NCHUNK + NS - 1) // NS):
        ch = sid + t * NS

        @pl.when(ch < NCHUNK)
        def _init_chunk():
            pltpu.sync_copy(qbuf, vs_sh.at[pl.ds(ch * C, C)])
            pltpu.sync_copy(zbuf, wn_sh.at[pl.ds(ch * C, C)])
    plsc.subcore_barrier()

    w = cid * NS + sid
    lanes = jnp.arange(16, dtype=jnp.int32)

    def _chunk(ci, _):
        off = w * EPW + ci * C
        pltpu.sync_copy(src_hbm.at[pl.ds(off, C)], sidx)
        pltpu.sync_copy(tgt_hbm.at[pl.ds(off, C)], tidx)
        cp_q = pltpu.async_copy(q_hbm.at[sidx], qbuf, sem)
        cp_k = pltpu.async_copy(k_hbm.at[tidx], kbuf, sem)
        cp_v = pltpu.async_copy(v_hbm.at[tidx], vbuf, sem)
        cp_q.wait()
        cp_k.wait()
        cp_v.wait()

        def _group(g, _):
            ev = lanes + g * 16
            for h in range(H):
                acc = zero16
                for dd in range(DH):
                    col = jnp.full((16,), h * DH + dd, jnp.int32)
                    qg = plsc.load_gather(qbuf, [ev, col])
                    kg = plsc.load_gather(kbuf, [ev, col])
                    acc = acc + qg * kg
                z = jnp.exp(acc * 0.25)
                plsc.store_scatter(zbuf, [ev, jnp.full((16,), h, jnp.int32)], z)
            return 0
        lax.fori_loop(0, C // 16, _group, 0)

        pltpu.sync_copy(zbuf, wn_sh.at[sidx], add=True)
        pltpu.sync_copy(vbuf, vs_sh.at[sidx], add=True)
        return 0

    lax.fori_loop(0, NCHUNK, _chunk, 0)
    plsc.subcore_barrier()

    @pl.when(sid != NS - 1)
    def _drain_main():
        pltpu.sync_copy(wn_sh.at[pl.ds(sid * DRA, DRA)],
                        wn_out.at[cid, pl.ds(sid * DRA, DRA)])
        pltpu.sync_copy(vs_sh.at[pl.ds(sid * DRA, DRA)],
                        vs_out.at[cid, pl.ds(sid * DRA, DRA)])

    @pl.when(sid == NS - 1)
    def _drain_last():
        pltpu.sync_copy(wn_sh.at[pl.ds((NS - 1) * DRA, DRB)],
                        wn_out.at[cid, pl.ds((NS - 1) * DRA, DRB)])
        pltpu.sync_copy(vs_sh.at[pl.ds((NS - 1) * DRA, DRB)],
                        vs_out.at[cid, pl.ds((NS - 1) * DRA, DRB)])


def _edge_pass(q, k, v, src, tgt):
    mesh = plsc.VectorSubcoreMesh(core_axis_name="c", subcore_axis_name="s")
    fn = pl.kernel(
        _edge_body,
        out_type=(
            jax.ShapeDtypeStruct((NC, N, 16), jnp.float32),
            jax.ShapeDtypeStruct((NC, N, D), jnp.float32),
        ),
        mesh=mesh,
        compiler_params=pltpu.CompilerParams(needs_layout_passes=False,
                                             use_tc_tiling_on_sc=False),
        scratch_types=(
            pltpu.VMEM((C,), jnp.int32),
            pltpu.VMEM((C,), jnp.int32),
            pltpu.VMEM((C, D), jnp.float32),
            pltpu.VMEM((C, D), jnp.float32),
            pltpu.VMEM((C, D), jnp.float32),
            pltpu.VMEM((C, 16), jnp.float32),
            pltpu.VMEM_SHARED((N, 16), jnp.float32),
            pltpu.VMEM_SHARED((N, D), jnp.float32),
            pltpu.SemaphoreType.DMA,
        ),
    )
    return fn(q, k, v, src, tgt)


def _combine_body(x_ref, wnf_ref, wnb_ref, vsb_ref, w1t_ref, b1_ref,
                  w2t_ref, b2_ref, g1_ref, bb1_ref, g2_ref, bb2_ref, y_ref):
    wn_all = wnf_ref[0] + wnf_ref[1]
    zh = jnp.sum(wn_all, axis=0, keepdims=True)
    zh = jnp.maximum(zh, 1e-30)
    attn = (wnb_ref[0] + wnb_ref[1]) / zh
    r16 = lax.broadcasted_iota(jnp.int32, (16, D), 0)
    c16 = lax.broadcasted_iota(jnp.int32, (16, D), 1)
    expand = (c16 // DH == r16).astype(jnp.float32)
    att128 = jnp.dot(attn, expand, preferred_element_type=jnp.float32)
    vs = vsb_ref[0] + vsb_ref[1]
    out = att128 * vs
    xb = x_ref[...]

    def _ln(t, g, b):
        mu = jnp.mean(t, axis=-1, keepdims=True)
        var = jnp.mean((t - mu) ** 2, axis=-1, keepdims=True)
        return (t - mu) * lax.rsqrt(var + 1e-5) * g + b

    h1 = _ln(xb + out, g1_ref[...], bb1_ref[...])
    f = jnp.maximum(jnp.dot(h1, w1t_ref[...],
                            preferred_element_type=jnp.float32) + b1_ref[...], 0.0)
    f = jnp.dot(f, w2t_ref[...], preferred_element_type=jnp.float32) + b2_ref[...]
    y_ref[...] = _ln(h1 + f, g2_ref[...], bb2_ref[...])


def _combine(x2, wn_p, vs_p, w1t, b1, w2t, b2, g1, bb1, g2, bb2):
    grid = N // BN
    full = lambda shape: pl.BlockSpec(shape, lambda i: tuple(0 for _ in shape))
    return pl.pallas_call(
        _combine_body,
        grid=(grid,),
        in_specs=[
            pl.BlockSpec((BN, D), lambda i: (i, 0)),
            full((NC, N, 16)),
            pl.BlockSpec((NC, BN, 16), lambda i: (0, i, 0)),
            pl.BlockSpec((NC, BN, D), lambda i: (0, i, 0)),
            full((D, 4 * D)),
            full((1, 4 * D)),
            full((4 * D, D)),
            full((1, D)),
            full((1, D)),
            full((1, D)),
            full((1, D)),
            full((1, D)),
        ],
        out_specs=pl.BlockSpec((BN, D), lambda i: (i, 0)),
        out_shape=jax.ShapeDtypeStruct((N, D), jnp.float32),
    )(x2, wn_p, wn_p, vs_p, w1t, b1, w2t, b2, g1, bb1, g2, bb2)


def kernel(x, node_indices, src, tgt, W_qkv, b_qkv, W1, b1, W2, b2,
           ln1_g, ln1_b, ln2_g, ln2_b):
    x2 = x.reshape(N, D)
    q, k, v = _qkv_project(x2, W_qkv.T, b_qkv.reshape(1, 3 * D))
    wn_p, vs_p = _edge_pass(q, k, v, src, tgt)
    y = _combine(x2, wn_p, vs_p, W1.T, b1.reshape(1, 4 * D), W2.T,
                 b2.reshape(1, D), ln1_g.reshape(1, D), ln1_b.reshape(1, D),
                 ln2_g.reshape(1, D), ln2_b.reshape(1, D))
    return y.reshape(1, N, D)
